# Initial kernel scaffold; baseline (speedup 1.0000x reference)
import jax
import jax.numpy as jnp
from jax.experimental import pallas as pl


def kernel(xyz, wbounds, embeddings):
    raise NotImplementedError("write your pallas kernel here")


# speedup vs baseline: 1.1703x; 1.1703x over previous
import functools

import numpy as np
import jax
import jax.numpy as jnp
from jax import lax
from jax.experimental import pallas as pl
from jax.experimental.pallas import tpu as pltpu
from jax.experimental.pallas import tpu_sc as plsc

EPS = 1e-6
NUM_LEVELS = 16
LEVEL_DIM = 2
B = 131072
PRIME1 = 2654435761
PRIME2 = 805459861

NC = 2
NS = 16
NW = NC * NS
CHUNK = 1024
NCHUNK = B // (NW * CHUNK)
NG = CHUNK // 16


def _level_offsets():
    offs, off = [], 0
    for i in range(NUM_LEVELS):
        res = int(np.ceil(16 * 2.0 ** i))
        params = min(2 ** 19, (res + 1) ** 3)
        params = int(params / 8) * 8
        offs.append(off)
        off += params
    offs.append(off)
    return offs


_OFFS = _level_offsets()
_SIZES = [_OFFS[i + 1] - _OFFS[i] for i in range(NUM_LEVELS)]
_SCALES = [float(np.exp2(i) * 16 - 1.0) for i in range(NUM_LEVELS)]


def _umod(h, m):
    if m & (m - 1) == 0:
        return h & jnp.uint32(m - 1)
    hi = lax.bitcast_convert_type(h >> jnp.uint32(1), jnp.int32).astype(jnp.float32)
    lo = lax.bitcast_convert_type(h & jnp.uint32(1), jnp.int32).astype(jnp.float32)
    hf = hi * 2.0 + lo
    q = (hf * float(1.0 / m)).astype(jnp.int32)
    r = h - lax.bitcast_convert_type(q, jnp.uint32) * jnp.uint32(m)
    r = jnp.where(lax.bitcast_convert_type(r, jnp.int32) < 0, r + jnp.uint32(m), r)
    r = jnp.where(lax.bitcast_convert_type(r, jnp.int32) >= m, r - jnp.uint32(m), r)
    return r


def _sc_body(xyzt, emb, out, xv, fv, outb, idxb, rows, sem):
    wid = lax.axis_index("s") * NC + lax.axis_index("c")
    pts = lax.iota(jnp.int32, 16)
    col0 = jnp.zeros((16,), jnp.int32)
    col1 = jnp.ones((16,), jnp.int32)

    def chunk_body(ck, carry):
        base = (wid * NCHUNK + ck) * CHUNK
        for d in range(3):
            pltpu.sync_copy(
                xyzt.at[pl.ds(d * B + base, CHUNK)], xv.at[pl.ds(d * CHUNK, CHUNK)]
            )

        for lvl in range(NUM_LEVELS):
            scale = _SCALES[lvl]
            msize = _SIZES[lvl]
            off = _OFFS[lvl]

            def pass_a(g, c, scale=scale, msize=msize, off=off):
                s = g * 16
                pg = []
                for d in range(3):
                    x = xv[pl.ds(d * CHUNK + s, 16)]
                    pos = x * scale + 0.5
                    pgi = pos.astype(jnp.int32)
                    fv[pl.ds(d * CHUNK + s, 16)] = pos - pgi.astype(jnp.float32)
                    pg.append(lax.bitcast_convert_type(pgi, jnp.uint32))
                x0, y0, z0 = pg
                xs = (x0, x0 + jnp.uint32(1))
                ys = (y0 * jnp.uint32(PRIME1), (y0 + jnp.uint32(1)) * jnp.uint32(PRIME1))
                zs = (z0 * jnp.uint32(PRIME2), (z0 + jnp.uint32(1)) * jnp.uint32(PRIME2))
                for corner in range(8):
                    h = xs[corner & 1] ^ ys[(corner >> 1) & 1] ^ zs[(corner >> 2) & 1]
                    row = _umod(h, msize) + jnp.uint32(off)
                    idxb[pl.ds(g * 128 + corner * 16, 16)] = lax.bitcast_convert_type(
                        row, jnp.int32
                    )
                pltpu.make_async_copy(
                    emb.at[idxb.at[pl.ds(g * 128, 128)]],
                    rows.at[pl.ds(g * 128, 128)],
                    sem,
                ).start()
                return c

            lax.fori_loop(0, NG, pass_a, 0)

            def drain(g, c):
                pltpu.make_async_copy(
                    emb.at[idxb.at[pl.ds(g * 128, 128)]],
                    rows.at[pl.ds(g * 128, 128)],
                    sem,
                ).wait()
                return c

            lax.fori_loop(0, NG, drain, 0)

            def pass_b(g, c, lvl=lvl):
                s = g * 16
                fx = fv[pl.ds(s, 16)]
                fy = fv[pl.ds(CHUNK + s, 16)]
                fz = fv[pl.ds(2 * CHUNK + s, 16)]
                ws = ((1.0 - fx, fx), (1.0 - fy, fy), (1.0 - fz, fz))
                acc0 = jnp.zeros((16,), jnp.float32)
                acc1 = jnp.zeros((16,), jnp.float32)
                rbase = g * 128
                for corner in range(8):
                    bx, by, bz = corner & 1, (corner >> 1) & 1, (corner >> 2) & 1
                    w = (ws[0][bx] * ws[1][by]) * ws[2][bz]
                    ridx = rbase + corner * 16 + pts
                    v0 = plsc.load_gather(rows, [ridx, col0])
                    v1 = plsc.load_gather(rows, [ridx, col1])
                    acc0 = acc0 + w * v0
                    acc1 = acc1 + w * v1
                plsc.store_scatter(outb, [s + pts, jnp.full((16,), 2 * lvl, jnp.int32)], acc0)
                plsc.store_scatter(outb, [s + pts, jnp.full((16,), 2 * lvl + 1, jnp.int32)], acc1)
                return c

            lax.fori_loop(0, NG, pass_b, 0)

        pltpu.sync_copy(outb, out.at[pl.ds(base, CHUNK)])
        return carry

    lax.fori_loop(0, NCHUNK, chunk_body, 0)


@jax.jit
def _hash_encode_sc(xyzt, emb):
    mesh = plsc.VectorSubcoreMesh(core_axis_name="c", subcore_axis_name="s")
    fn = functools.partial(
        pl.kernel,
        out_type=jax.ShapeDtypeStruct((B, NUM_LEVELS * LEVEL_DIM), jnp.float32),
        mesh=mesh,
        compiler_params=pltpu.CompilerParams(
            needs_layout_passes=False, use_tc_tiling_on_sc=False
        ),
        scratch_types=[
            pltpu.VMEM((3 * CHUNK,), jnp.float32),
            pltpu.VMEM((3 * CHUNK,), jnp.float32),
            pltpu.VMEM((CHUNK, NUM_LEVELS * LEVEL_DIM), jnp.float32),
            pltpu.VMEM((NG * 128,), jnp.int32),
            pltpu.VMEM((NG * 128, LEVEL_DIM), jnp.float32),
            pltpu.SemaphoreType.DMA,
        ],
    )(_sc_body)
    return fn(xyzt, emb)


def kernel(xyz, wbounds, embeddings):
    inputs = jnp.clip(xyz, wbounds[:3], wbounds[3:6])
    inputs = inputs - wbounds[None, :3]
    inputs = inputs / (jnp.max(wbounds[3:6] - wbounds[:3]) + EPS)
    return _hash_encode_sc(inputs.T.reshape(-1), embeddings)
